# Initial kernel scaffold; baseline (speedup 1.0000x reference)
import jax
import jax.numpy as jnp
from jax.experimental import pallas as pl


def kernel(aa, pos14, atom_mask, residual_table, atom_table):
    raise NotImplementedError("write your pallas kernel here")


# speedup vs baseline: 5.5871x; 5.5871x over previous
import functools

import jax
import jax.numpy as jnp
from jax import lax
from jax.experimental import pallas as pl
from jax.experimental.pallas import tpu as pltpu
from jax.experimental.pallas import tpu_sc as plsc

N, L, HALF = 16, 2048, 64
A = 14
D = A * 2 * HALF
B = N * L
R = 21

NW = 32
B_PER_W = B // NW
CHUNK = 16
NBUF = 4
NCH = B_PER_W // CHUNK
NITER = NCH // NBUF


def _build_comb_kernel(rt_ref, at_ref, out_ref):
    rt = rt_ref[...]
    at = at_ref[...]
    out_ref[...] = jnp.concatenate(
        [
            jnp.broadcast_to(rt[:, None, :], (R, A, HALF)),
            jnp.broadcast_to(at[None, :, :], (R, A, HALF)),
        ],
        axis=-1,
    )


def _make_gather_kernel():
    mesh = plsc.VectorSubcoreMesh(core_axis_name="c", subcore_axis_name="s")
    scratch = [pltpu.VMEM((NCH, CHUNK), jnp.int32)]
    scratch += [pltpu.VMEM((CHUNK, D), jnp.float32) for _ in range(NBUF)]
    scratch += [pltpu.SemaphoreType.DMA for _ in range(2 * NBUF)]

    @functools.partial(
        pl.kernel,
        mesh=mesh,
        out_type=jax.ShapeDtypeStruct((B, D), jnp.float32),
        scratch_types=scratch,
    )
    def gather_kernel(idx_hbm, table_hbm, out_hbm, idx_v, *rest):
        bufs = rest[:NBUF]
        gsems = rest[NBUF:2 * NBUF]
        ssems = rest[2 * NBUF:]
        wid = lax.axis_index("s") * 2 + lax.axis_index("c")
        base = wid * B_PER_W
        pltpu.sync_copy(idx_hbm.at[wid], idx_v)

        def start_gather(c, b):
            pltpu.async_copy(table_hbm.at[idx_v.at[c]], bufs[b], gsems[b])

        def wait_gather(c, b):
            pltpu.make_async_copy(
                table_hbm.at[idx_v.at[c]], bufs[b], gsems[b]).wait()

        def start_scatter(c, b):
            pltpu.async_copy(
                bufs[b], out_hbm.at[pl.ds(base + c * CHUNK, CHUNK)], ssems[b])

        def wait_scatter(c, b):
            pltpu.make_async_copy(
                bufs[b], out_hbm.at[pl.ds(base + c * CHUNK, CHUNK)],
                ssems[b]).wait()

        for b in range(NBUF):
            start_gather(b, b)

        def body(i, carry):
            c0 = i * NBUF
            for b in range(NBUF):
                wait_gather(c0 + b, b)
                start_scatter(c0 + b, b)
            for b in range(NBUF):
                wait_scatter(c0 + b, b)
                start_gather(c0 + NBUF + b, b)
            return carry

        lax.fori_loop(0, NITER - 1, body, 0)

        c0 = (NITER - 1) * NBUF
        for b in range(NBUF):
            wait_gather(c0 + b, b)
            start_scatter(c0 + b, b)
        for b in range(NBUF):
            wait_scatter(c0 + b, b)

    return gather_kernel


_GATHER = _make_gather_kernel()


def kernel(aa, pos14, atom_mask, residual_table, atom_table):
    comb3 = pl.pallas_call(
        _build_comb_kernel,
        out_shape=jax.ShapeDtypeStruct((R, A, 2 * HALF), jnp.float32),
    )(residual_table, atom_table)
    comb = comb3.reshape(R, D)
    idx = aa.astype(jnp.int32).reshape(NW, NCH, CHUNK)
    out = _GATHER(idx, comb)
    feats = out.reshape(N, L * A, 2 * HALF)
    coors = pos14.reshape(N, L * A, 3)
    mask = atom_mask.reshape(N, L * A)
    return (feats, coors, mask)


# speedup vs baseline: 5.6584x; 1.0128x over previous
import functools

import jax
import jax.numpy as jnp
from jax import lax
from jax.experimental import pallas as pl
from jax.experimental.pallas import tpu as pltpu
from jax.experimental.pallas import tpu_sc as plsc

N, L, HALF = 16, 2048, 64
A = 14
D = A * 2 * HALF
B = N * L
R = 21

NW = 32
B_PER_W = B // NW
CHUNK = 8
NBUF = 8
NCH = B_PER_W // CHUNK
NITER = NCH // NBUF


def _build_comb_kernel(rt_ref, at_ref, out_ref):
    rt = rt_ref[...]
    at = at_ref[...]
    out_ref[...] = jnp.concatenate(
        [
            jnp.broadcast_to(rt[:, None, :], (R, A, HALF)),
            jnp.broadcast_to(at[None, :, :], (R, A, HALF)),
        ],
        axis=-1,
    )


def _make_gather_kernel():
    mesh = plsc.VectorSubcoreMesh(core_axis_name="c", subcore_axis_name="s")
    scratch = [pltpu.VMEM((NCH, CHUNK), jnp.int32)]
    scratch += [pltpu.VMEM((CHUNK, D), jnp.float32) for _ in range(NBUF)]
    scratch += [pltpu.SemaphoreType.DMA for _ in range(2 * NBUF)]

    @functools.partial(
        pl.kernel,
        mesh=mesh,
        out_type=jax.ShapeDtypeStruct((B, D), jnp.float32),
        scratch_types=scratch,
    )
    def gather_kernel(idx_hbm, table_hbm, out_hbm, idx_v, *rest):
        bufs = rest[:NBUF]
        gsems = rest[NBUF:2 * NBUF]
        ssems = rest[2 * NBUF:]
        wid = lax.axis_index("s") * 2 + lax.axis_index("c")
        base = wid * B_PER_W
        pltpu.sync_copy(idx_hbm.at[wid], idx_v)

        def start_gather(c, b):
            pltpu.async_copy(table_hbm.at[idx_v.at[c]], bufs[b], gsems[b])

        def wait_gather(c, b):
            pltpu.make_async_copy(
                table_hbm.at[idx_v.at[c]], bufs[b], gsems[b]).wait()

        def start_scatter(c, b):
            pltpu.async_copy(
                bufs[b], out_hbm.at[pl.ds(base + c * CHUNK, CHUNK)], ssems[b])

        def wait_scatter(c, b):
            pltpu.make_async_copy(
                bufs[b], out_hbm.at[pl.ds(base + c * CHUNK, CHUNK)],
                ssems[b]).wait()

        for b in range(NBUF):
            start_gather(b, b)

        def body(i, carry):
            c0 = i * NBUF
            for b in range(NBUF):
                wait_gather(c0 + b, b)
                start_scatter(c0 + b, b)
            for b in range(NBUF):
                wait_scatter(c0 + b, b)
                start_gather(c0 + NBUF + b, b)
            return carry

        lax.fori_loop(0, NITER - 1, body, 0)

        c0 = (NITER - 1) * NBUF
        for b in range(NBUF):
            wait_gather(c0 + b, b)
            start_scatter(c0 + b, b)
        for b in range(NBUF):
            wait_scatter(c0 + b, b)

    return gather_kernel


_GATHER = _make_gather_kernel()


def kernel(aa, pos14, atom_mask, residual_table, atom_table):
    comb3 = pl.pallas_call(
        _build_comb_kernel,
        out_shape=jax.ShapeDtypeStruct((R, A, 2 * HALF), jnp.float32),
    )(residual_table, atom_table)
    comb = comb3.reshape(R, D)
    idx = aa.astype(jnp.int32).reshape(NW, NCH, CHUNK)
    out = _GATHER(idx, comb)
    feats = out.reshape(N, L * A, 2 * HALF)
    coors = pos14.reshape(N, L * A, 3)
    mask = atom_mask.reshape(N, L * A)
    return (feats, coors, mask)


# speedup vs baseline: 8.1611x; 1.4423x over previous
import functools

import jax
import jax.numpy as jnp
from jax import lax
from jax.experimental import pallas as pl
from jax.experimental.pallas import tpu as pltpu
from jax.experimental.pallas import tpu_sc as plsc

N, L, HALF = 16, 2048, 64
A = 14
DF = 2 * HALF
B = N * L * A
R = 21

NW = 32
B_PER_W = B // NW
CHUNK = 64
NBUF = 8
NCH = B_PER_W // CHUNK
NITER = NCH // NBUF


def _build_table2_kernel(rt_ref, at_ref, out_ref):
    rt = rt_ref[...]
    at = at_ref[...]
    out_ref[...] = jnp.concatenate(
        [
            jnp.broadcast_to(rt[:, None, :], (R, A, HALF)),
            jnp.broadcast_to(at[None, :, :], (R, A, HALF)),
        ],
        axis=-1,
    )


def _make_gather_kernel():
    mesh = plsc.VectorSubcoreMesh(core_axis_name="c", subcore_axis_name="s")
    scratch = [pltpu.VMEM((NCH, CHUNK), jnp.int32)]
    scratch += [pltpu.VMEM((CHUNK, DF), jnp.float32) for _ in range(NBUF)]
    scratch += [pltpu.SemaphoreType.DMA for _ in range(2 * NBUF)]

    @functools.partial(
        pl.kernel,
        mesh=mesh,
        out_type=jax.ShapeDtypeStruct((B, DF), jnp.float32),
        scratch_types=scratch,
    )
    def gather_kernel(idx_hbm, table_hbm, out_hbm, idx_v, *rest):
        bufs = rest[:NBUF]
        gsems = rest[NBUF:2 * NBUF]
        ssems = rest[2 * NBUF:]
        wid = lax.axis_index("s") * 2 + lax.axis_index("c")
        base = wid * B_PER_W
        pltpu.sync_copy(idx_hbm.at[wid], idx_v)

        def start_gather(c, b):
            pltpu.async_copy(table_hbm.at[idx_v.at[c]], bufs[b], gsems[b])

        def wait_gather(c, b):
            pltpu.make_async_copy(
                table_hbm.at[idx_v.at[c]], bufs[b], gsems[b]).wait()

        def start_scatter(c, b):
            pltpu.async_copy(
                bufs[b], out_hbm.at[pl.ds(base + c * CHUNK, CHUNK)], ssems[b])

        def wait_scatter(c, b):
            pltpu.make_async_copy(
                bufs[b], out_hbm.at[pl.ds(base + c * CHUNK, CHUNK)],
                ssems[b]).wait()

        for b in range(NBUF):
            start_gather(b, b)

        def body(i, carry):
            c0 = i * NBUF
            for b in range(NBUF):
                wait_gather(c0 + b, b)
                start_scatter(c0 + b, b)
            for b in range(NBUF):
                wait_scatter(c0 + b, b)
                start_gather(c0 + NBUF + b, b)
            return carry

        lax.fori_loop(0, NITER - 1, body, 0)

        c0 = (NITER - 1) * NBUF
        for b in range(NBUF):
            wait_gather(c0 + b, b)
            start_scatter(c0 + b, b)
        for b in range(NBUF):
            wait_scatter(c0 + b, b)

    return gather_kernel


_GATHER = _make_gather_kernel()


def kernel(aa, pos14, atom_mask, residual_table, atom_table):
    table3 = pl.pallas_call(
        _build_table2_kernel,
        out_shape=jax.ShapeDtypeStruct((R, A, DF), jnp.float32),
    )(residual_table, atom_table)
    table2 = table3.reshape(R * A, DF)
    aa32 = aa.astype(jnp.int32)
    idx = (aa32[:, :, None] * A
           + jnp.arange(A, dtype=jnp.int32)).reshape(NW, NCH, CHUNK)
    out = _GATHER(idx, table2)
    feats = out.reshape(N, L * A, DF)
    coors = pos14.reshape(N, L * A, 3)
    mask = atom_mask.reshape(N, L * A)
    return (feats, coors, mask)
